# Initial kernel scaffold; baseline (speedup 1.0000x reference)
import jax
import jax.numpy as jnp
from jax.experimental import pallas as pl


def kernel(image, direction, object_tab, color_tab, state_tab, direction_tab, position_tab, ln_gamma, ln_beta):
    raise NotImplementedError("write your pallas kernel here")


# speedup vs baseline: 1.8328x; 1.8328x over previous
import functools

import jax
import jax.numpy as jnp
from jax import lax
from jax.experimental import pallas as pl
from jax.experimental.pallas import tpu as pltpu
from jax.experimental.pallas import tpu_sc as plsc

L = 16
D = 64
HW = 256
NTOK = HW + 1
EPS = 1e-5


def _rsqrt(v):
    i = plsc.bitcast(v, jnp.int32)
    y = plsc.bitcast(jnp.int32(0x5F3759DF) - lax.shift_right_logical(i, 1),
                     jnp.float32)
    for _ in range(3):
        y = y * (1.5 - 0.5 * v * y * y)
    return y


def _body(img, dirr, oT, cT, sT, dirtab, posT, gamma, beta, out,
          oT_v, cT_v, sT_v, dirtab_v, oct_v, posT_v, dirln, gb_v, bb_v,
          imgb, dirb, tokbuf, abuf, bbuf, outb, isem0, isem1, osem0, osem1,
          *, items_per_worker):
    ix = lax.iota(jnp.int32, L)
    wid = lax.axis_index("s") * 2 + lax.axis_index("c")
    base = wid * items_per_worker
    row_elems = HW * 3

    def full(x):
        return jnp.full((L,), x, dtype=jnp.int32)

    pltpu.sync_copy(oT, oT_v)
    pltpu.sync_copy(cT, cT_v)
    pltpu.sync_copy(sT, sT_v)
    pltpu.sync_copy(dirtab, dirtab_v)
    pltpu.sync_copy(posT, posT_v)
    pltpu.sync_copy(gamma, gb_v)
    pltpu.sync_copy(beta, bb_v)
    pltpu.sync_copy(dirr.at[pl.ds(base, items_per_worker)], dirb)

    gq = [gb_v[pl.ds(q * L, L)] for q in range(4)]
    bq = [bb_v[pl.ds(q * L, L)] for q in range(4)]

    def build_oct(f, carry):
        cv = plsc.load_gather(cT_v, [f * 16 + ix])
        for i0 in range(16):
            osc = plsc.load_gather(oT_v, [full(f * 16 + i0)])
            plsc.store_scatter(oct_v, [f * 256 + i0 * 16 + ix], osc + cv)
        return carry
    lax.fori_loop(0, D, build_oct, 0)

    for d in range(4):
        xq = []
        for q in range(4):
            p256 = plsc.load_gather(posT_v, [(q * L + ix) * NTOK + HW])
            xq.append(dirtab_v[pl.ds(d * D + q * L, L)] + p256)
        s = xq[0] + xq[1] + xq[2] + xq[3]
        s2 = xq[0] * xq[0] + xq[1] * xq[1] + xq[2] * xq[2] + xq[3] * xq[3]
        mean = jnp.broadcast_to(jnp.sum(s), (L,)) * (1.0 / D)
        ex2 = jnp.broadcast_to(jnp.sum(s2), (L,)) * (1.0 / D)
        rstd = _rsqrt(ex2 - mean * mean + EPS)
        for q in range(4):
            dirln[pl.ds(d * D + q * L, L)] = (xq[q] - mean) * rstd * gq[q] + bq[q]

    pltpu.async_copy(img.at[pl.ds(base * row_elems, row_elems)],
                     imgb.at[pl.ds(0, row_elems)], isem0)
    pltpu.async_copy(img.at[pl.ds((base + 1) * row_elems, row_elems)],
                     imgb.at[pl.ds(row_elems, row_elems)], isem1)

    def item_step(it, carry):
        for k, (isem, osem) in ((0, (isem0, osem0)), (1, (isem1, osem1))):
            item = 2 * it + k
            islot = imgb.at[pl.ds(k * row_elems, row_elems)]
            oslot = outb.at[pl.ds(k * NTOK * D, NTOK * D)]
            pltpu.make_async_copy(img.at[pl.ds((base + item) * row_elems,
                                            row_elems)], islot, isem).wait()

            @pl.when(it >= 1)
            def _wait_out():
                pltpu.make_async_copy(
                    oslot, out.at[pl.ds((base + item - 2) * NTOK * D,
                                        NTOK * D)], osem).wait()

            def group(g, carry2):
                @pl.when(g < HW // L)
                def _phase_a():
                    g16 = g * L + ix
                    tok3 = k * row_elems + g16 * 3
                    i0 = plsc.load_gather(imgb, [tok3])
                    i1 = plsc.load_gather(imgb, [tok3 + 1])
                    i2 = plsc.load_gather(imgb, [tok3 + 2])
                    ci = i0 * 16 + i1
                    tslot = (g % 2) * (L * D)

                    def feat(fi, acc):
                        acc_s, acc_q = acc
                        for u in range(4):
                            fo = fi * 4 + u
                            x = (plsc.load_gather(oct_v, [full(fo * 256) + ci])
                                 + plsc.load_gather(sT_v, [full(fo * 16) + i2])
                                 + plsc.load_gather(
                                     posT_v, [full(fo * NTOK + g * L) + ix]))
                            acc_s = acc_s + x
                            acc_q = acc_q + x * x
                            plsc.store_scatter(tokbuf,
                                               [full(tslot + fo) + ix * D], x)
                        return acc_s, acc_q

                    zero = jnp.zeros((L,), jnp.float32)
                    acc_s, acc_q = lax.fori_loop(0, D // 4, feat, (zero, zero))
                    mean = acc_s * (1.0 / D)
                    rstd = _rsqrt(acc_q * (1.0 / D) - mean * mean + EPS)
                    abuf[pl.ds(g * L, L)] = rstd
                    bbuf[pl.ds(g * L, L)] = -mean * rstd

                @pl.when(g >= 1)
                def _phase_b():
                    gp = g - 1
                    tslot = (gp % 2) * (L * D)
                    obase = k * NTOK * D + gp * (L * D)
                    for t in range(L):
                        at = plsc.load_gather(abuf, [full(gp * L + t)])
                        bt = plsc.load_gather(bbuf, [full(gp * L + t)])
                        for q in range(4):
                            v = tokbuf[pl.ds(tslot + t * D + q * L, L)]
                            o = (v * at + bt) * gq[q] + bq[q]
                            plsc.store_scatter(
                                outb, [full(obase + t * D + q * L) + ix], o)
                return carry2

            lax.fori_loop(0, HW // L + 1, group, 0)

            dsp = plsc.load_gather(dirb, [full(item)])
            for q in range(4):
                vq = plsc.load_gather(dirln, [dsp * D + q * L + ix])
                plsc.store_scatter(
                    outb, [full(k * NTOK * D + HW * D + q * L) + ix], vq)

            pltpu.async_copy(oslot,
                             out.at[pl.ds((base + item) * NTOK * D, NTOK * D)],
                             osem)

            @pl.when(item + 2 < items_per_worker)
            def _prefetch():
                pltpu.async_copy(img.at[pl.ds((base + item + 2) * row_elems,
                                              row_elems)], islot, isem)
        return carry

    lax.fori_loop(0, items_per_worker // 2, item_step, 0)
    pltpu.make_async_copy(
        outb.at[pl.ds(0, NTOK * D)],
        out.at[pl.ds((base + items_per_worker - 2) * NTOK * D, NTOK * D)],
        osem0).wait()
    pltpu.make_async_copy(
        outb.at[pl.ds(NTOK * D, NTOK * D)],
        out.at[pl.ds((base + items_per_worker - 1) * NTOK * D, NTOK * D)],
        osem1).wait()


def kernel(image, direction, object_tab, color_tab, state_tab, direction_tab,
           position_tab, ln_gamma, ln_beta):
    b, h, w, _ = image.shape
    img2 = image.astype(jnp.int32).reshape(b * h * w * 3)
    d_i = jnp.clip(direction.astype(jnp.int32), 0, 3)
    n_workers = 32
    ipw = b // n_workers

    mesh = plsc.VectorSubcoreMesh(core_axis_name="c", subcore_axis_name="s",
                                  num_cores=2, num_subcores=16)
    f32 = jnp.float32
    run = pl.kernel(
        functools.partial(_body, items_per_worker=ipw),
        out_type=jax.ShapeDtypeStruct((b * NTOK * D,), f32),
        mesh=mesh,
        compiler_params=pltpu.CompilerParams(needs_layout_passes=False),
        scratch_types=[
            pltpu.VMEM((D * 16,), f32),
            pltpu.VMEM((D * 16,), f32),
            pltpu.VMEM((D * 16,), f32),
            pltpu.VMEM((4 * D,), f32),
            pltpu.VMEM((D * 256,), f32),
            pltpu.VMEM((D * NTOK,), f32),
            pltpu.VMEM((4 * D,), f32),
            pltpu.VMEM((D,), f32),
            pltpu.VMEM((D,), f32),
            pltpu.VMEM((2 * h * w * 3,), jnp.int32),
            pltpu.VMEM((b // n_workers,), jnp.int32),
            pltpu.VMEM((2 * L * D,), f32),
            pltpu.VMEM((HW,), f32),
            pltpu.VMEM((HW,), f32),
            pltpu.VMEM((2 * NTOK * D,), f32),
            pltpu.SemaphoreType.DMA,
            pltpu.SemaphoreType.DMA,
            pltpu.SemaphoreType.DMA,
            pltpu.SemaphoreType.DMA,
        ],
    )
    out = run(img2, d_i,
              object_tab.T.reshape(-1), color_tab.T.reshape(-1),
              state_tab.T.reshape(-1), direction_tab.reshape(-1),
              position_tab.T.reshape(-1), ln_gamma, ln_beta)
    return out.reshape(b, NTOK, D)


# speedup vs baseline: 2.0958x; 1.1435x over previous
import functools

import jax
import jax.numpy as jnp
from jax import lax
from jax.experimental import pallas as pl
from jax.experimental.pallas import tpu as pltpu
from jax.experimental.pallas import tpu_sc as plsc

L = 16
D = 64
HW = 256
NTOK = HW + 1
EPS = 1e-5


def _rsqrt(v):
    i = plsc.bitcast(v, jnp.int32)
    y = plsc.bitcast(jnp.int32(0x5F3759DF) - lax.shift_right_logical(i, 1),
                     jnp.float32)
    for _ in range(3):
        y = y * (1.5 - 0.5 * v * y * y)
    return y


def _body(img, dirr, oT, cT, sT, dirtab, posT, gamma, beta, out,
          oT_v, cT_v, sT_v, dirtab_v, oct_v, posT_v, dirln, gb_v, bb_v,
          imgb, dirb, tokbuf, abuf, bbuf, outb, isem0, isem1, osem0, osem1,
          *, items_per_worker):
    ix = lax.iota(jnp.int32, L)
    wid = lax.axis_index("s") * 2 + lax.axis_index("c")
    base = wid * items_per_worker
    row_elems = HW * 3

    def full(x):
        return jnp.full((L,), x, dtype=jnp.int32)

    pltpu.sync_copy(oT, oT_v)
    pltpu.sync_copy(cT, cT_v)
    pltpu.sync_copy(sT, sT_v)
    pltpu.sync_copy(dirtab, dirtab_v)
    pltpu.sync_copy(posT, posT_v)
    pltpu.sync_copy(gamma, gb_v)
    pltpu.sync_copy(beta, bb_v)
    pltpu.sync_copy(dirr.at[pl.ds(base, items_per_worker)], dirb)

    gq = [gb_v[pl.ds(q * L, L)] for q in range(4)]
    bq = [bb_v[pl.ds(q * L, L)] for q in range(4)]

    def build_oct(f, carry):
        cv = plsc.load_gather(cT_v, [f * 16 + ix])
        for i0 in range(16):
            osc = plsc.load_gather(oT_v, [full(f * 16 + i0)])
            plsc.store_scatter(oct_v, [f * 256 + i0 * 16 + ix], osc + cv)
        return carry
    lax.fori_loop(0, D, build_oct, 0)

    for d in range(4):
        xq = []
        for q in range(4):
            p256 = plsc.load_gather(posT_v, [(q * L + ix) * NTOK + HW])
            xq.append(dirtab_v[pl.ds(d * D + q * L, L)] + p256)
        s = xq[0] + xq[1] + xq[2] + xq[3]
        s2 = xq[0] * xq[0] + xq[1] * xq[1] + xq[2] * xq[2] + xq[3] * xq[3]
        mean = jnp.broadcast_to(jnp.sum(s), (L,)) * (1.0 / D)
        ex2 = jnp.broadcast_to(jnp.sum(s2), (L,)) * (1.0 / D)
        rstd = _rsqrt(ex2 - mean * mean + EPS)
        for q in range(4):
            dirln[pl.ds(d * D + q * L, L)] = (xq[q] - mean) * rstd * gq[q] + bq[q]

    pltpu.async_copy(img.at[pl.ds(base * row_elems, row_elems)],
                     imgb.at[pl.ds(0, row_elems)], isem0)
    pltpu.async_copy(img.at[pl.ds((base + 1) * row_elems, row_elems)],
                     imgb.at[pl.ds(row_elems, row_elems)], isem1)

    def item_step(it, carry):
        for k, (isem, osem) in ((0, (isem0, osem0)), (1, (isem1, osem1))):
            item = 2 * it + k
            islot = imgb.at[pl.ds(k * row_elems, row_elems)]
            oslot = outb.at[pl.ds(k * NTOK * D, NTOK * D)]
            pltpu.make_async_copy(img.at[pl.ds((base + item) * row_elems,
                                            row_elems)], islot, isem).wait()

            @pl.when(it >= 1)
            def _wait_out():
                pltpu.make_async_copy(
                    oslot, out.at[pl.ds((base + item - 2) * NTOK * D,
                                        NTOK * D)], osem).wait()

            def group(g, carry2):
                @pl.when(g < HW // L)
                def _phase_a():
                    g16 = g * L + ix
                    tok3 = k * row_elems + g16 * 3
                    i0 = plsc.load_gather(imgb, [tok3])
                    i1 = plsc.load_gather(imgb, [tok3 + 1])
                    i2 = plsc.load_gather(imgb, [tok3 + 2])
                    ci = i0 * 16 + i1
                    tslot = (g % 2) * (L * 65)

                    def feat(fi, acc):
                        acc_s, acc_q = acc
                        for u in range(4):
                            fo = fi * 4 + u
                            x = (plsc.load_gather(oct_v, [full(fo * 256) + ci])
                                 + plsc.load_gather(sT_v, [full(fo * 16) + i2])
                                 + plsc.load_gather(
                                     posT_v, [full(fo * NTOK + g * L) + ix]))
                            acc_s = acc_s + x
                            acc_q = acc_q + x * x
                            plsc.store_scatter(tokbuf,
                                               [full(tslot + fo) + ix * 65], x)
                        return acc_s, acc_q

                    zero = jnp.zeros((L,), jnp.float32)
                    acc_s, acc_q = lax.fori_loop(0, D // 4, feat, (zero, zero))
                    mean = acc_s * (1.0 / D)
                    rstd = _rsqrt(acc_q * (1.0 / D) - mean * mean + EPS)
                    abuf[pl.ds(g * L, L)] = rstd
                    bbuf[pl.ds(g * L, L)] = -mean * rstd

                @pl.when(g >= 1)
                def _phase_b():
                    gp = g - 1
                    tslot = (gp % 2) * (L * 65)
                    obase = k * NTOK * D + gp * (L * D)
                    for t in range(L):
                        at = plsc.load_gather(abuf, [full(gp * L + t)])
                        bt = plsc.load_gather(bbuf, [full(gp * L + t)])
                        for q in range(4):
                            v = tokbuf[pl.ds(tslot + t * 65 + q * L, L)]
                            o = (v * at + bt) * gq[q] + bq[q]
                            plsc.store_scatter(
                                outb, [full(obase + t * D + q * L) + ix], o)
                return carry2

            lax.fori_loop(0, HW // L + 1, group, 0)

            dsp = plsc.load_gather(dirb, [full(item)])
            for q in range(4):
                vq = plsc.load_gather(dirln, [dsp * D + q * L + ix])
                plsc.store_scatter(
                    outb, [full(k * NTOK * D + HW * D + q * L) + ix], vq)

            pltpu.async_copy(oslot,
                             out.at[pl.ds((base + item) * NTOK * D, NTOK * D)],
                             osem)

            @pl.when(item + 2 < items_per_worker)
            def _prefetch():
                pltpu.async_copy(img.at[pl.ds((base + item + 2) * row_elems,
                                              row_elems)], islot, isem)
        return carry

    lax.fori_loop(0, items_per_worker // 2, item_step, 0)
    pltpu.make_async_copy(
        outb.at[pl.ds(0, NTOK * D)],
        out.at[pl.ds((base + items_per_worker - 2) * NTOK * D, NTOK * D)],
        osem0).wait()
    pltpu.make_async_copy(
        outb.at[pl.ds(NTOK * D, NTOK * D)],
        out.at[pl.ds((base + items_per_worker - 1) * NTOK * D, NTOK * D)],
        osem1).wait()


def kernel(image, direction, object_tab, color_tab, state_tab, direction_tab,
           position_tab, ln_gamma, ln_beta):
    b, h, w, _ = image.shape
    img2 = image.astype(jnp.int32).reshape(b * h * w * 3)
    d_i = jnp.clip(direction.astype(jnp.int32), 0, 3)
    n_workers = 32
    ipw = b // n_workers

    mesh = plsc.VectorSubcoreMesh(core_axis_name="c", subcore_axis_name="s",
                                  num_cores=2, num_subcores=16)
    f32 = jnp.float32
    run = pl.kernel(
        functools.partial(_body, items_per_worker=ipw),
        out_type=jax.ShapeDtypeStruct((b * NTOK * D,), f32),
        mesh=mesh,
        compiler_params=pltpu.CompilerParams(needs_layout_passes=False),
        scratch_types=[
            pltpu.VMEM((D * 16,), f32),
            pltpu.VMEM((D * 16,), f32),
            pltpu.VMEM((D * 16,), f32),
            pltpu.VMEM((4 * D,), f32),
            pltpu.VMEM((D * 256,), f32),
            pltpu.VMEM((D * NTOK,), f32),
            pltpu.VMEM((4 * D,), f32),
            pltpu.VMEM((D,), f32),
            pltpu.VMEM((D,), f32),
            pltpu.VMEM((2 * h * w * 3,), jnp.int32),
            pltpu.VMEM((b // n_workers,), jnp.int32),
            pltpu.VMEM((2 * L * 65,), f32),
            pltpu.VMEM((HW,), f32),
            pltpu.VMEM((HW,), f32),
            pltpu.VMEM((2 * NTOK * D,), f32),
            pltpu.SemaphoreType.DMA,
            pltpu.SemaphoreType.DMA,
            pltpu.SemaphoreType.DMA,
            pltpu.SemaphoreType.DMA,
        ],
    )
    out = run(img2, d_i,
              object_tab.T.reshape(-1), color_tab.T.reshape(-1),
              state_tab.T.reshape(-1), direction_tab.reshape(-1),
              position_tab.T.reshape(-1), ln_gamma, ln_beta)
    return out.reshape(b, NTOK, D)


# speedup vs baseline: 2.7546x; 1.3143x over previous
import jax
import jax.numpy as jnp
from jax import lax
from jax.experimental import pallas as pl
from jax.experimental.pallas import tpu as pltpu
from jax.experimental.pallas import tpu_sc as plsc

L = 16
D = 64
HW = 256
NTOK = HW + 1
EPS = 1e-5
NW = 32
BPW = 128
PSTR = 129


def _sc_body(img, dirr, oT, cT, sT, dirtabT, out,
             oT_v, cT_v, sT_v, dirtabT_v, oct_v, strep, dirb, pidx,
             imgb, outb, osem0, osem1):
    ix = lax.iota(jnp.int32, L)
    wid = lax.axis_index("s") * 2 + lax.axis_index("c")
    base_b = wid * BPW

    def full(x):
        return jnp.full((L,), x, dtype=jnp.int32)

    pltpu.sync_copy(oT, oT_v)
    pltpu.sync_copy(cT, cT_v)
    pltpu.sync_copy(sT, sT_v)
    pltpu.sync_copy(dirtabT, dirtabT_v)
    pltpu.sync_copy(dirr.at[pl.ds(base_b, BPW)], dirb)

    def build_oct(f, carry):
        cv = plsc.load_gather(cT_v, [f * 16 + ix])
        for i0 in range(16):
            osc = plsc.load_gather(oT_v, [full(f * 16 + i0)])
            plsc.store_scatter(oct_v, [f * 256 + i0 * 16 + ix], osc + cv)
        return carry
    lax.fori_loop(0, D, build_oct, 0)

    def build_strep(f, carry):
        sv = plsc.load_gather(sT_v, [f * 16 + ix])
        for r in range(16):
            plsc.store_scatter(strep, [full(f * 256 + r) + ix * 16], sv)
        return carry
    lax.fori_loop(0, D, build_strep, 0)

    def build_pidx(item, carry):
        pltpu.sync_copy(img.at[pl.ds((base_b + item) * (HW * 3), HW * 3)],
                        imgb)

        def grp(g, c2):
            g16 = g * L + ix
            tok3 = g16 * 3
            i0 = plsc.load_gather(imgb, [tok3])
            i1 = plsc.load_gather(imgb, [tok3 + 1])
            i2 = plsc.load_gather(imgb, [tok3 + 2])
            pk = i0 * 16 + i1 + i2 * 256
            plsc.store_scatter(pidx, [g16 * PSTR + item], pk)
            return c2
        lax.fori_loop(0, HW // L, grp, 0)
        return carry
    lax.fori_loop(0, BPW, build_pidx, 0)

    def pos_step(it, carry):
        for k, osem in ((0, osem0), (1, osem1)):
            p = 2 * it + k
            koff = k * (D * BPW)

            @pl.when(jnp.logical_and(it >= 1, p - 2 < NTOK))
            def _wait_out():
                pltpu.make_async_copy(
                    outb.at[pl.ds(koff, D * BPW)],
                    out.at[pl.ds(((p - 2) * NW + wid) * (D * BPW), D * BPW)],
                    osem).wait()

            @pl.when(p < HW)
            def _tokens():
                pks = [plsc.load_gather(pidx, [p * PSTR + bg * L + ix])
                       for bg in range(8)]
                cis = [pk & 255 for pk in pks]
                sbs = [lax.shift_right_logical(pk, 8) * 16 + ix for pk in pks]

                def feat(fi, c2):
                    for u in range(4):
                        fo = fi * 4 + u
                        fo256 = full(fo * 256)
                        for bg in range(8):
                            x = (plsc.load_gather(oct_v, [fo256 + cis[bg]])
                                 + plsc.load_gather(strep, [fo256 + sbs[bg]]))
                            outb[pl.ds(koff + fo * BPW + bg * L, L)] = x
                    return c2
                lax.fori_loop(0, D // 4, feat, 0)

            @pl.when(p == HW)
            def _dir_row():
                dgs = [plsc.load_gather(dirb, [bg * L + ix])
                       for bg in range(8)]

                def featd(fi, c2):
                    for u in range(4):
                        fo = fi * 4 + u
                        fo4 = full(fo * 4)
                        for bg in range(8):
                            x = plsc.load_gather(dirtabT_v, [fo4 + dgs[bg]])
                            outb[pl.ds(koff + fo * BPW + bg * L, L)] = x
                    return c2
                lax.fori_loop(0, D // 4, featd, 0)

            @pl.when(p < NTOK)
            def _flush():
                pltpu.async_copy(
                    outb.at[pl.ds(koff, D * BPW)],
                    out.at[pl.ds((p * NW + wid) * (D * BPW), D * BPW)],
                    osem)
        return carry

    lax.fori_loop(0, (NTOK + 1) // 2 + 1, pos_step, 0)


def _tc_body(x_ref, pos_ref, g_ref, b_ref, o_ref):
    v = x_ref[...]
    pos = pos_ref[...]
    v = v + pos.reshape(1, D, 1)
    m = jnp.mean(v, axis=1, keepdims=True)
    var = jnp.mean(v * v, axis=1, keepdims=True) - m * m
    y = (v - m) * lax.rsqrt(var + EPS)
    y = y * g_ref[...].reshape(1, D, 1) + b_ref[...].reshape(1, D, 1)
    o_ref[...] = jnp.transpose(y, (1, 0, 2)).reshape(1, D, NW * BPW)


def kernel(image, direction, object_tab, color_tab, state_tab, direction_tab,
           position_tab, ln_gamma, ln_beta):
    b, h, w, _ = image.shape
    f32 = jnp.float32
    img2 = image.astype(jnp.int32).reshape(b * h * w * 3)
    d_i = jnp.clip(direction.astype(jnp.int32), 0, 3)

    mesh = plsc.VectorSubcoreMesh(core_axis_name="c", subcore_axis_name="s",
                                  num_cores=2, num_subcores=16)
    sc = pl.kernel(
        _sc_body,
        out_type=jax.ShapeDtypeStruct((NTOK * NW * D * BPW,), f32),
        mesh=mesh,
        compiler_params=pltpu.CompilerParams(needs_layout_passes=False),
        scratch_types=[
            pltpu.VMEM((D * 16,), f32),
            pltpu.VMEM((D * 16,), f32),
            pltpu.VMEM((D * 16,), f32),
            pltpu.VMEM((D * 4,), f32),
            pltpu.VMEM((D * 256,), f32),
            pltpu.VMEM((D * 256,), f32),
            pltpu.VMEM((BPW,), jnp.int32),
            pltpu.VMEM((HW * PSTR,), jnp.int32),
            pltpu.VMEM((HW * 3,), jnp.int32),
            pltpu.VMEM((2 * D * BPW,), f32),
            pltpu.SemaphoreType.DMA,
            pltpu.SemaphoreType.DMA,
        ],
    )
    tok = sc(img2, d_i,
             object_tab.T.reshape(-1), color_tab.T.reshape(-1),
             state_tab.T.reshape(-1), direction_tab.T.reshape(-1))
    tok3 = tok.reshape(NTOK * NW, D, BPW)

    ln = pl.pallas_call(
        _tc_body,
        grid=(NTOK,),
        in_specs=[
            pl.BlockSpec((NW, D, BPW), lambda i: (i, 0, 0)),
            pl.BlockSpec((1, 1, D), lambda i: (i, 0, 0)),
            pl.BlockSpec((1, D), lambda i: (0, 0)),
            pl.BlockSpec((1, D), lambda i: (0, 0)),
        ],
        out_specs=pl.BlockSpec((1, D, NW * BPW), lambda i: (i, 0, 0)),
        out_shape=jax.ShapeDtypeStruct((NTOK, D, b), f32),
    )(tok3, position_tab.reshape(NTOK, 1, D),
      ln_gamma.reshape(1, D), ln_beta.reshape(1, D))

    return jnp.transpose(ln, (2, 0, 1))
